# Initial kernel scaffold; baseline (speedup 1.0000x reference)
import jax
import jax.numpy as jnp
from jax.experimental import pallas as pl


def kernel(features, coords, W_ft, b_ft, W_coord, b_coord, W_feat, b_feat, ln_g, ln_b):
    raise NotImplementedError("write your pallas kernel here")


# speedup vs baseline: 5.6480x; 5.6480x over previous
import functools

import jax
import jax.numpy as jnp
import numpy as np
from jax import lax
from jax.experimental import pallas as pl
from jax.experimental.pallas import tpu as pltpu
from jax.experimental.pallas import tpu_sc as plsc

N_PTS = 10000
CH = 128
KNN = 8
NPAD = 10240
ROWS_A = 128
ROWS_C = 128
TBLW = 256
BIGF = 3.0e38
BIGI = 2**30


def _knn_body(a_ref, b_ref, idx_ref):
    p = jnp.dot(a_ref[:, 0:3].astype(jnp.bfloat16),
                b_ref[0:3, :].astype(jnp.bfloat16),
                preferred_element_type=jnp.float32)
    d = a_ref[:, 3:4] + b_ref[3:4, :] - 2.0 * p
    iota = lax.broadcasted_iota(jnp.int32, (ROWS_A, NPAD), 1)
    cols = []
    for _ in range(KNN):
        m = jnp.min(d, axis=1, keepdims=True)
        cand = jnp.where(d == m, iota, BIGI)
        j = jnp.min(cand, axis=1, keepdims=True)
        cols.append(j)
        d = jnp.where(iota == j, BIGF, d)
    idx_ref[...] = jnp.concatenate(cols, axis=1)


def _attn_body(tbl_ref, g_ref, wc_ref, wf_ref, wt_ref, vecs_ref, out_ref):
    x = tbl_ref[:, 0:CH]
    xc = tbl_ref[:, CH:CH + 8]
    b_coord = vecs_ref[0:1, :]
    b_feat = vecs_ref[1:2, :]
    b_ft = vecs_ref[2:3, :]
    ln_g = vecs_ref[3:4, :]
    ln_b = vecs_ref[4:5, :]
    scale = float(np.sqrt(float(KNN)))

    scores = []
    trans = []
    for k in range(KNN):
        nf = g_ref[k, :, 0:CH]
        nc = g_ref[k, :, CH:CH + 8]
        rel_c = nc - xc
        rel_f = nf - x
        s = (jnp.dot(rel_c, wc_ref[...], preferred_element_type=jnp.float32)
             + b_coord) * \
            (jnp.dot(rel_f, wf_ref[...], preferred_element_type=jnp.float32)
             + b_feat) / scale
        t = jnp.dot(nf, wt_ref[...], preferred_element_type=jnp.float32) + b_ft
        scores.append(s)
        trans.append(t)

    m = scores[0]
    for k in range(1, KNN):
        m = jnp.maximum(m, scores[k])
    es = [jnp.exp(s - m) for s in scores]
    z = es[0]
    for k in range(1, KNN):
        z = z + es[k]
    upd = (es[0] / z) * trans[0]
    for k in range(1, KNN):
        upd = upd + (es[k] / z) * trans[k]

    out = upd + x
    mu = jnp.mean(out, axis=-1, keepdims=True)
    var = jnp.mean((out - mu) ** 2, axis=-1, keepdims=True)
    out_ref[...] = (out - mu) / jnp.sqrt(var + 1e-5) * ln_g + ln_b


def _knn_call(a_mat, b_mat):
    grid = NPAD // ROWS_A
    return pl.pallas_call(
        _knn_body,
        grid=(grid,),
        in_specs=[
            pl.BlockSpec((ROWS_A, 8), lambda i: (i, 0)),
            pl.BlockSpec((8, NPAD), lambda i: (0, 0)),
        ],
        out_specs=pl.BlockSpec((ROWS_A, KNN), lambda i: (i, 0)),
        out_shape=jax.ShapeDtypeStruct((NPAD, KNN), jnp.int32),
    )(a_mat, b_mat)


def _attn_call(table, g3, wc, wf, wt, vecs):
    grid = NPAD // ROWS_C
    return pl.pallas_call(
        _attn_body,
        grid=(grid,),
        in_specs=[
            pl.BlockSpec((ROWS_C, TBLW), lambda i: (i, 0)),
            pl.BlockSpec((KNN, ROWS_C, TBLW), lambda i: (0, i, 0)),
            pl.BlockSpec((8, CH), lambda i: (0, 0)),
            pl.BlockSpec((CH, CH), lambda i: (0, 0)),
            pl.BlockSpec((CH, CH), lambda i: (0, 0)),
            pl.BlockSpec((8, CH), lambda i: (0, 0)),
        ],
        out_specs=pl.BlockSpec((ROWS_C, CH), lambda i: (i, 0)),
        out_shape=jax.ShapeDtypeStruct((NPAD, CH), jnp.float32),
    )(table, g3, wc, wf, wt, vecs)


_NW = 32
_GB = KNN * NPAD
_B_PER_W = _GB // _NW
_CHUNK = 128
_NCH = _B_PER_W // _CHUNK


def _gather_rows(table, idx_flat):
    mesh = plsc.VectorSubcoreMesh(core_axis_name="c", subcore_axis_name="s")

    @functools.partial(
        pl.kernel, mesh=mesh,
        out_type=jax.ShapeDtypeStruct((_GB, TBLW), jnp.float32),
        scratch_types=[
            pltpu.VMEM((_B_PER_W,), jnp.int32),
            pltpu.VMEM((_CHUNK, TBLW), jnp.float32),
            pltpu.VMEM((_CHUNK, TBLW), jnp.float32),
            pltpu.SemaphoreType.DMA,
            pltpu.SemaphoreType.DMA,
        ],
    )
    def gk(table_hbm, idx_hbm, out_hbm, idx_v, r0, r1, s0, s1):
        wid = lax.axis_index("s") * 2 + lax.axis_index("c")
        base = wid * _B_PER_W
        pltpu.sync_copy(idx_hbm.at[pl.ds(base, _B_PER_W)], idx_v)
        bufs = (r0, r1)
        sems = (s0, s1)
        cps = [None, None]
        for c in range(_NCH):
            b = c % 2
            cps[b] = pltpu.async_copy(
                table_hbm.at[idx_v.at[pl.ds(c * _CHUNK, _CHUNK)]],
                bufs[b], sems[b])
            if c > 0:
                pb = (c - 1) % 2
                cps[pb].wait()
                pltpu.sync_copy(
                    bufs[pb],
                    out_hbm.at[pl.ds(base + (c - 1) * _CHUNK, _CHUNK)])
        lb = (_NCH - 1) % 2
        cps[lb].wait()
        pltpu.sync_copy(
            bufs[lb], out_hbm.at[pl.ds(base + (_NCH - 1) * _CHUNK, _CHUNK)])

    return gk(table, idx_flat)


def kernel(features, coords, W_ft, b_ft, W_coord, b_coord, W_feat, b_feat,
           ln_g, ln_b):
    bc = coords[:, 1:]
    sq = jnp.sum(bc * bc, axis=-1)
    pad = NPAD - N_PTS

    bc_p = jnp.pad(bc, ((0, pad), (0, 0)))
    sq_p = jnp.pad(sq, (0, pad))
    feat_p = jnp.pad(features, ((0, pad), (0, 0)))

    zeros4 = jnp.zeros((NPAD, 4), jnp.float32)
    a_mat = jnp.concatenate([bc_p, sq_p[:, None], zeros4], axis=1)
    col_mask = jnp.where(jnp.arange(NPAD) >= N_PTS, jnp.float32(1e30),
                         jnp.float32(0.0))
    b_mat = jnp.concatenate(
        [bc_p.T, (sq_p + col_mask)[None, :], zeros4.T], axis=0)

    idx = _knn_call(a_mat, b_mat)
    idx_flat = idx.T.reshape(-1)

    table = jnp.concatenate(
        [feat_p, bc_p, jnp.zeros((NPAD, TBLW - CH - 3), jnp.float32)], axis=1)
    g = _gather_rows(table, idx_flat)
    g3 = g.reshape(KNN, NPAD, TBLW)

    wc = jnp.concatenate([W_coord, jnp.zeros((5, CH), jnp.float32)], axis=0)
    vecs = jnp.stack([b_coord, b_feat, b_ft, ln_g, ln_b,
                      jnp.zeros_like(b_ft), jnp.zeros_like(b_ft),
                      jnp.zeros_like(b_ft)], axis=0)

    out = _attn_call(table, g3, wc, W_feat, W_ft, vecs)
    return out[:N_PTS]


# speedup vs baseline: 6.5044x; 1.1516x over previous
import functools

import jax
import jax.numpy as jnp
import numpy as np
from jax import lax
from jax.experimental import pallas as pl
from jax.experimental.pallas import tpu as pltpu
from jax.experimental.pallas import tpu_sc as plsc

N_PTS = 10000
CH = 128
KNN = 8
NPAD = 10240
ROWS_A = 128
ROWS_C = 128
TBLW = 256
BIGF = 3.0e38
BIGI = 2**30


def _knn_body(a_ref, b_ref, idx_ref):
    p = jnp.dot(a_ref[:, 0:3].astype(jnp.bfloat16),
                b_ref[0:3, :].astype(jnp.bfloat16),
                preferred_element_type=jnp.float32)
    d = a_ref[:, 3:4] + b_ref[3:4, :] - 2.0 * p
    iota = lax.broadcasted_iota(jnp.int32, (ROWS_A, NPAD), 1).astype(
        jnp.float32)
    cols = []
    for _ in range(KNN):
        m = jnp.min(d, axis=1, keepdims=True)
        j = jnp.min(jnp.where(d == m, iota, BIGF), axis=1, keepdims=True)
        cols.append(j.astype(jnp.int32))
        d = jnp.where(iota == j, BIGF, d)
    idx_ref[...] = jnp.concatenate(cols, axis=1)


def _attn_body(tbl_ref, g_ref, wc_ref, wf_ref, wt_ref, vecs_ref, out_ref):
    x = tbl_ref[:, 0:CH]
    xc = tbl_ref[:, CH:CH + 8]
    b_coord = vecs_ref[0:1, :]
    b_feat = vecs_ref[1:2, :]
    b_ft = vecs_ref[2:3, :]
    ln_g = vecs_ref[3:4, :]
    ln_b = vecs_ref[4:5, :]
    scale = float(np.sqrt(float(KNN)))

    scores = []
    trans = []
    for k in range(KNN):
        nf = g_ref[k, :, 0:CH]
        nc = g_ref[k, :, CH:CH + 8]
        rel_c = nc - xc
        rel_f = nf - x
        s = (jnp.dot(rel_c, wc_ref[...], preferred_element_type=jnp.float32)
             + b_coord) * \
            (jnp.dot(rel_f, wf_ref[...], preferred_element_type=jnp.float32)
             + b_feat) / scale
        t = jnp.dot(nf, wt_ref[...], preferred_element_type=jnp.float32) + b_ft
        scores.append(s)
        trans.append(t)

    m = scores[0]
    for k in range(1, KNN):
        m = jnp.maximum(m, scores[k])
    es = [jnp.exp(s - m) for s in scores]
    z = es[0]
    for k in range(1, KNN):
        z = z + es[k]
    upd = (es[0] / z) * trans[0]
    for k in range(1, KNN):
        upd = upd + (es[k] / z) * trans[k]

    out = upd + x
    mu = jnp.mean(out, axis=-1, keepdims=True)
    var = jnp.mean((out - mu) ** 2, axis=-1, keepdims=True)
    out_ref[...] = (out - mu) / jnp.sqrt(var + 1e-5) * ln_g + ln_b


def _knn_call(a_mat, b_mat):
    grid = NPAD // ROWS_A
    return pl.pallas_call(
        _knn_body,
        grid=(grid,),
        in_specs=[
            pl.BlockSpec((ROWS_A, 8), lambda i: (i, 0)),
            pl.BlockSpec((8, NPAD), lambda i: (0, 0)),
        ],
        out_specs=pl.BlockSpec((ROWS_A, KNN), lambda i: (i, 0)),
        out_shape=jax.ShapeDtypeStruct((NPAD, KNN), jnp.int32),
    )(a_mat, b_mat)


def _attn_call(table, g3, wc, wf, wt, vecs):
    grid = NPAD // ROWS_C
    return pl.pallas_call(
        _attn_body,
        grid=(grid,),
        in_specs=[
            pl.BlockSpec((ROWS_C, TBLW), lambda i: (i, 0)),
            pl.BlockSpec((KNN, ROWS_C, TBLW), lambda i: (0, i, 0)),
            pl.BlockSpec((8, CH), lambda i: (0, 0)),
            pl.BlockSpec((CH, CH), lambda i: (0, 0)),
            pl.BlockSpec((CH, CH), lambda i: (0, 0)),
            pl.BlockSpec((8, CH), lambda i: (0, 0)),
        ],
        out_specs=pl.BlockSpec((ROWS_C, CH), lambda i: (i, 0)),
        out_shape=jax.ShapeDtypeStruct((NPAD, CH), jnp.float32),
    )(table, g3, wc, wf, wt, vecs)


_NW = 32
_GB = KNN * NPAD
_B_PER_W = _GB // _NW
_CHUNK = 128
_NCH = _B_PER_W // _CHUNK


def _gather_rows(table, idx_flat):
    mesh = plsc.VectorSubcoreMesh(core_axis_name="c", subcore_axis_name="s")

    @functools.partial(
        pl.kernel, mesh=mesh,
        out_type=jax.ShapeDtypeStruct((_GB, TBLW), jnp.float32),
        scratch_types=[
            pltpu.VMEM((_B_PER_W,), jnp.int32),
            pltpu.VMEM((_CHUNK, TBLW), jnp.float32),
            pltpu.VMEM((_CHUNK, TBLW), jnp.float32),
            pltpu.SemaphoreType.DMA,
            pltpu.SemaphoreType.DMA,
        ],
    )
    def gk(table_hbm, idx_hbm, out_hbm, idx_v, r0, r1, s0, s1):
        wid = lax.axis_index("s") * 2 + lax.axis_index("c")
        base = wid * _B_PER_W
        pltpu.sync_copy(idx_hbm.at[pl.ds(base, _B_PER_W)], idx_v)
        bufs = (r0, r1)
        sems = (s0, s1)
        cps = [None, None]
        for c in range(_NCH):
            b = c % 2
            cps[b] = pltpu.async_copy(
                table_hbm.at[idx_v.at[pl.ds(c * _CHUNK, _CHUNK)]],
                bufs[b], sems[b])
            if c > 0:
                pb = (c - 1) % 2
                cps[pb].wait()
                pltpu.sync_copy(
                    bufs[pb],
                    out_hbm.at[pl.ds(base + (c - 1) * _CHUNK, _CHUNK)])
        lb = (_NCH - 1) % 2
        cps[lb].wait()
        pltpu.sync_copy(
            bufs[lb], out_hbm.at[pl.ds(base + (_NCH - 1) * _CHUNK, _CHUNK)])

    return gk(table, idx_flat)


def kernel(features, coords, W_ft, b_ft, W_coord, b_coord, W_feat, b_feat,
           ln_g, ln_b):
    bc = coords[:, 1:]
    sq = jnp.sum(bc * bc, axis=-1)
    pad = NPAD - N_PTS

    bc_p = jnp.pad(bc, ((0, pad), (0, 0)))
    sq_p = jnp.pad(sq, (0, pad))
    feat_p = jnp.pad(features, ((0, pad), (0, 0)))

    zeros4 = jnp.zeros((NPAD, 4), jnp.float32)
    a_mat = jnp.concatenate([bc_p, sq_p[:, None], zeros4], axis=1)
    col_mask = jnp.where(jnp.arange(NPAD) >= N_PTS, jnp.float32(1e30),
                         jnp.float32(0.0))
    b_mat = jnp.concatenate(
        [bc_p.T, (sq_p + col_mask)[None, :], zeros4.T], axis=0)

    idx = _knn_call(a_mat, b_mat)
    idx_flat = idx.T.reshape(-1)

    table = jnp.concatenate(
        [feat_p, bc_p, jnp.zeros((NPAD, TBLW - CH - 3), jnp.float32)], axis=1)
    g = _gather_rows(table, idx_flat)
    g3 = g.reshape(KNN, NPAD, TBLW)

    wc = jnp.concatenate([W_coord, jnp.zeros((5, CH), jnp.float32)], axis=0)
    vecs = jnp.stack([b_coord, b_feat, b_ft, ln_g, ln_b,
                      jnp.zeros_like(b_ft), jnp.zeros_like(b_ft),
                      jnp.zeros_like(b_ft)], axis=0)

    out = _attn_call(table, g3, wc, W_feat, W_ft, vecs)
    return out[:N_PTS]
